# baseline (speedup 1.0000x reference)
import jax
import jax.numpy as jnp
from jax.experimental import pallas as pl
from jax.experimental.pallas import tpu as pltpu

_D = 64
_C = 16


def _mlp_kernel(x_ref, w1_ref, bmat_ref, wo2_ref, bcat_ref, out_ref):
    x0 = x_ref[...]
    old = jnp.dot(x0, w1_ref[...], preferred_element_type=jnp.float32)
    old = jnp.maximum(old + bcat_ref[:, 0:_D], 0.0)
    mid = jnp.dot(old, bmat_ref[...], preferred_element_type=jnp.float32)
    new = jnp.maximum(mid[:, 0:_D] + bcat_ref[:, _D:2 * _D], 0.0)
    logits = (
        mid[:, _D:]
        + jnp.dot(new, wo2_ref[...], preferred_element_type=jnp.float32)
        + bcat_ref[:, 2 * _D:]
    )
    e = jnp.exp(logits)
    out_ref[...] = e / jnp.sum(e, axis=-1, keepdims=True)


def kernel(x_feats, edge_index, agg_W, agg_b, W1, b1, W2, b2, W_out, b_out):
    del edge_index, agg_W, agg_b
    B, _, D = x_feats.shape
    C = W_out.shape[1]
    x0 = jax.lax.slice_in_dim(x_feats, 0, 1, axis=1).reshape(B, D)
    bmat = jnp.concatenate([W2, W_out[:D]], axis=1)
    bcat = jnp.concatenate([b1, b2, b_out]).reshape(1, -1)
    return pl.pallas_call(
        _mlp_kernel,
        out_shape=jax.ShapeDtypeStruct((B, C), jnp.float32),
        compiler_params=pltpu.CompilerParams(
            allow_input_fusion=[True] * 5,
        ),
    )(x0, W1, bmat, W_out[D:], bcat)


# speedup vs baseline: 1.3166x; 1.3166x over previous
import jax
import jax.numpy as jnp
from jax.experimental import pallas as pl
from jax.experimental.pallas import tpu as pltpu

_D = 64


def _mlp_kernel(x_ref, w1_ref, b1_ref, w2_ref, b2_ref, wo_ref, bo_ref, out_ref):
    x0 = x_ref[:, 0, :]
    old = jnp.dot(x0, w1_ref[...], preferred_element_type=jnp.float32)
    old = jnp.maximum(old + b1_ref[...], 0.0)
    new = jnp.dot(old, w2_ref[...], preferred_element_type=jnp.float32)
    new = jnp.maximum(new + b2_ref[...], 0.0)
    logits = (
        jnp.dot(old, wo_ref[:_D, :], preferred_element_type=jnp.float32)
        + jnp.dot(new, wo_ref[_D:, :], preferred_element_type=jnp.float32)
        + bo_ref[...]
    )
    e = jnp.exp(logits)
    out_ref[...] = e / jnp.sum(e, axis=-1, keepdims=True)


def kernel(x_feats, edge_index, agg_W, agg_b, W1, b1, W2, b2, W_out, b_out):
    del edge_index, agg_W, agg_b
    B, _, D = x_feats.shape
    C = W_out.shape[1]
    x0 = jax.lax.slice_in_dim(x_feats, 0, 1, axis=1)
    return pl.pallas_call(
        _mlp_kernel,
        out_shape=jax.ShapeDtypeStruct((B, C), jnp.float32),
        compiler_params=pltpu.CompilerParams(
            allow_input_fusion=[True] * 7,
        ),
    )(x0, W1, b1.reshape(1, D), W2, b2.reshape(1, D), W_out, b_out.reshape(1, C))


# speedup vs baseline: 1.3237x; 1.0054x over previous
import jax
import jax.numpy as jnp
from jax.experimental import pallas as pl
from jax.experimental.pallas import tpu as pltpu

_D = 64


def _mlp_kernel(x_ref, w1_ref, b1_ref, w2_ref, b2_ref, wo_ref, bo_ref, out_ref):
    x0 = x_ref[...]
    old = jnp.dot(x0, w1_ref[...], preferred_element_type=jnp.float32)
    old = jnp.maximum(old + b1_ref[...], 0.0)
    new = jnp.dot(old, w2_ref[...], preferred_element_type=jnp.float32)
    new = jnp.maximum(new + b2_ref[...], 0.0)
    logits = (
        jnp.dot(old, wo_ref[:_D, :], preferred_element_type=jnp.float32)
        + jnp.dot(new, wo_ref[_D:, :], preferred_element_type=jnp.float32)
        + bo_ref[...]
    )
    e = jnp.exp(logits)
    out_ref[...] = e / jnp.sum(e, axis=-1, keepdims=True)


def kernel(x_feats, edge_index, agg_W, agg_b, W1, b1, W2, b2, W_out, b_out):
    del edge_index, agg_W, agg_b
    B, _, D = x_feats.shape
    C = W_out.shape[1]
    x0 = jax.lax.slice_in_dim(x_feats, 0, 1, axis=1).reshape(B, D)
    return pl.pallas_call(
        _mlp_kernel,
        out_shape=jax.ShapeDtypeStruct((B, C), jnp.float32),
        compiler_params=pltpu.CompilerParams(
            allow_input_fusion=[True] * 7,
        ),
    )(x0, W1, b1, W2, b2, W_out, b_out)
